# Initial kernel scaffold; baseline (speedup 1.0000x reference)
import jax
import jax.numpy as jnp
from jax.experimental import pallas as pl


def kernel(x, edge_index, edge_weight):
    raise NotImplementedError("write your pallas kernel here")


# speedup vs baseline: 594.9835x; 594.9835x over previous
import functools

import jax
import jax.numpy as jnp
from jax import lax
from jax.experimental import pallas as pl
from jax.experimental.pallas import tpu as pltpu
from jax.experimental.pallas import tpu_sc as plsc

N_NODES = 10000
N_EDGES = 320000
D_FEAT = 128
D_HALF = 64
LANES = 16

NUM_CORES = 2
NUM_SUBCORES = 16
NUM_WORKERS = NUM_CORES * NUM_SUBCORES

SUB = 100
SUBS_PER_MACRO = 8
EDGE_ROWS = 2 * N_EDGES // SUB
ROWS_PER_TILE = EDGE_ROWS // NUM_WORKERS
MACROS_PER_TILE = ROWS_PER_TILE // SUBS_PER_MACRO

N_PAD = 10240
NODES_PER_TILE = N_PAD // NUM_SUBCORES
LAST_TILE_VALID = N_NODES - 15 * (N_PAD // NUM_SUBCORES)


def _sc_body(xcat_hbm, gidx_hbm, sidx_hbm, w_hbm, out_hbm,
             acc, cnt, gbuf, sbuf, wbuf, rows, ones, zbuf, zcnt, sem):
    c = lax.axis_index("c")
    s = lax.axis_index("s")
    wid = c * NUM_SUBCORES + s

    zv = jnp.zeros((LANES,), jnp.float32)
    ov = jnp.ones((LANES,), jnp.float32)

    def init_ones(i, carry):
        ones[i, :] = ov
        return carry
    lax.fori_loop(0, SUB, init_ones, 0)

    def init_z(i, carry):
        for k in range(D_HALF // LANES):
            zbuf[i, pl.ds(k * LANES, LANES)] = zv
        zcnt[i, :] = zv
        return carry
    lax.fori_loop(0, NODES_PER_TILE, init_z, 0)

    node0 = s * NODES_PER_TILE
    pltpu.sync_copy(zbuf, acc.at[pl.ds(node0, NODES_PER_TILE)])
    pltpu.sync_copy(zcnt, cnt.at[pl.ds(node0, NODES_PER_TILE)])
    plsc.subcore_barrier()

    row0 = wid * ROWS_PER_TILE

    def macro_body(m, carry):
        r0 = row0 + m * SUBS_PER_MACRO
        pltpu.sync_copy(gidx_hbm.at[pl.ds(r0, SUBS_PER_MACRO)], gbuf)
        pltpu.sync_copy(sidx_hbm.at[pl.ds(r0, SUBS_PER_MACRO)], sbuf)
        pltpu.sync_copy(w_hbm.at[pl.ds(r0, SUBS_PER_MACRO)], wbuf)

        def sub_body(j, inner):
            pltpu.async_copy(xcat_hbm.at[gbuf.at[j]], rows, sem).wait()

            def mul_body(g, acc_):
                wv = wbuf[j, pl.ds(g * LANES, LANES)]
                for l in range(LANES):
                    w = wv[l]
                    e = g * LANES + l
                    for k in range(D_HALF // LANES):
                        sl = pl.ds(k * LANES, LANES)
                        rows[e, sl] = rows[e, sl] * w
                return acc_
            lax.fori_loop(0, SUB // LANES, mul_body, 0)
            wt = wbuf[j, pl.ds(96, 4)]
            for l in range(4):
                w = wt[l]
                for k in range(D_HALF // LANES):
                    sl = pl.ds(k * LANES, LANES)
                    rows[96 + l, sl] = rows[96 + l, sl] * w

            pltpu.sync_copy(rows, acc.at[sbuf.at[j]], add=True)
            pltpu.sync_copy(ones, cnt.at[sbuf.at[j]], add=True)
            return inner
        lax.fori_loop(0, SUBS_PER_MACRO, sub_body, 0)
        return carry
    lax.fori_loop(0, MACROS_PER_TILE, macro_body, 0)

    plsc.subcore_barrier()

    pltpu.sync_copy(acc.at[pl.ds(node0, NODES_PER_TILE)], zbuf)
    pltpu.sync_copy(cnt.at[pl.ds(node0, NODES_PER_TILE)], zcnt)

    def norm_body(i, carry):
        invv = ov / jnp.maximum(zcnt[i, :], 1.0)
        for k in range(D_HALF // LANES):
            sl = pl.ds(k * LANES, LANES)
            zbuf[i, sl] = zbuf[i, sl] * invv
        return carry
    lax.fori_loop(0, NODES_PER_TILE, norm_body, 0)

    @pl.when(s < NUM_SUBCORES - 1)
    def _write_full():
        pltpu.sync_copy(zbuf, out_hbm.at[c, pl.ds(node0, NODES_PER_TILE)])

    @pl.when(s == NUM_SUBCORES - 1)
    def _write_tail():
        pltpu.sync_copy(zbuf.at[pl.ds(0, LAST_TILE_VALID)],
                        out_hbm.at[c, pl.ds(node0, LAST_TILE_VALID)])


@jax.jit
def _prop_conv_sc(xcat, gidx, sidx, wts):
    mesh = plsc.VectorSubcoreMesh(core_axis_name="c", subcore_axis_name="s")
    fn = functools.partial(
        pl.kernel,
        mesh=mesh,
        compiler_params=pltpu.CompilerParams(use_tc_tiling_on_sc=False),
        out_type=jax.ShapeDtypeStruct((NUM_CORES, N_NODES, D_HALF), jnp.float32),
        scratch_types=[
            pltpu.VMEM_SHARED((N_PAD, D_HALF), jnp.float32),
            pltpu.VMEM_SHARED((N_PAD, LANES), jnp.float32),
            pltpu.VMEM((SUBS_PER_MACRO, SUB), jnp.int32),
            pltpu.VMEM((SUBS_PER_MACRO, SUB), jnp.int32),
            pltpu.VMEM((SUBS_PER_MACRO, SUB), jnp.float32),
            pltpu.VMEM((SUB, D_HALF), jnp.float32),
            pltpu.VMEM((SUB, LANES), jnp.float32),
            pltpu.VMEM((NODES_PER_TILE, D_HALF), jnp.float32),
            pltpu.VMEM((NODES_PER_TILE, LANES), jnp.float32),
            pltpu.SemaphoreType.DMA,
        ],
    )(_sc_body)
    return fn(xcat, gidx, sidx, wts)


def kernel(x, edge_index, edge_weight):
    x = x.astype(jnp.float32)
    row = edge_index[0].astype(jnp.int32)
    col = edge_index[1].astype(jnp.int32)
    w = edge_weight.astype(jnp.float32)

    xcat = jnp.concatenate([x[:, :D_HALF], x[:, D_HALF:]], axis=0)
    gidx = jnp.concatenate([col, row + N_NODES]).reshape(EDGE_ROWS, SUB)
    sidx = jnp.concatenate([row, col]).reshape(EDGE_ROWS, SUB)
    wts = jnp.concatenate([w, w]).reshape(EDGE_ROWS, SUB)
    out = _prop_conv_sc(xcat, gidx, sidx, wts)
    return jnp.concatenate([out[0], out[1]], axis=-1)
